# baseline (speedup 1.0000x reference)
import functools

import jax
import jax.numpy as jnp
from jax import lax
from jax.experimental import pallas as pl
from jax.experimental.pallas import tpu as pltpu
from jax.experimental.pallas import tpu_sc as plsc

L = 16
NIN = 3
NOUT = 3


def _sc_add_kernel(B, S, D, R, n_workers):
    s_per_w = S // n_workers
    n_blocks = s_per_w // R
    H = R // 2
    assert S % n_workers == 0 and s_per_w % R == 0 and R % 2 == 0
    assert D % L == 0 and n_blocks > NIN
    n_main = n_blocks - (n_blocks % NIN or NIN)
    tail = n_blocks - n_main

    mesh = plsc.VectorSubcoreMesh(core_axis_name="c", subcore_axis_name="s")

    scratch = []
    for _ in range(NIN):
        scratch.append(pltpu.VMEM((R, D), jnp.float32))
        scratch.append(pltpu.VMEM((B, R, D), jnp.float32))
        scratch.append(pltpu.SemaphoreType.DMA)
        scratch.append(pltpu.SemaphoreType.DMA)
    for _ in range(NOUT):
        scratch.append(pltpu.VMEM((B, H, D), jnp.float32))
        scratch.append(pltpu.SemaphoreType.DMA)

    @functools.partial(
        pl.kernel,
        mesh=mesh,
        out_type=jax.ShapeDtypeStruct((B, S, D), jnp.float32),
        scratch_types=scratch,
    )
    def k(x_hbm, p_hbm, o_hbm, *bufs):
        ins = [bufs[4 * i: 4 * i + 4] for i in range(NIN)]
        outs = [bufs[4 * NIN + 2 * i: 4 * NIN + 2 * i + 2] for i in range(NOUT)]
        wid = lax.axis_index("c") * 16 + lax.axis_index("s")
        base0 = wid * s_per_w

        def fire_in(blk, st):
            pbuf, xbuf, semp, semx = st
            base = base0 + blk * R
            pltpu.async_copy(p_hbm.at[pl.ds(base, R)], pbuf, semp)
            pltpu.async_copy(x_hbm.at[:, pl.ds(base, R)], xbuf, semx)

        def drain_half(ob):
            hbuf, semo = ob
            pltpu.make_async_copy(
                hbuf, o_hbm.at[:, pl.ds(base0, H)], semo).wait()

        def process(blk, t):
            pbuf, xbuf, semp, semx = ins[t]
            base = base0 + blk * R

            @pl.when(blk + 2 < n_blocks)
            def _():
                fire_in(blk + 2, ins[(t + 2) % NIN])

            pltpu.make_async_copy(p_hbm.at[pl.ds(base, R)], pbuf, semp).wait()
            pltpu.make_async_copy(x_hbm.at[:, pl.ds(base, R)], xbuf, semx).wait()

            for q in range(2):
                ob = outs[(2 * t + q) % NOUT]
                hbuf, semo = ob
                hbase = base + q * H

                @pl.when(2 * blk + q >= NOUT)
                def _():
                    drain_half(ob)

                @plsc.parallel_loop(0, D // L, unroll=2)
                def _(i):
                    c = i * L
                    for r in range(H):
                        pv = pbuf[q * H + r, pl.ds(c, L)]
                        for b in range(B):
                            hbuf[b, r, pl.ds(c, L)] = (
                                xbuf[b, q * H + r, pl.ds(c, L)] + pv)

                pltpu.async_copy(hbuf, o_hbm.at[:, pl.ds(hbase, H)], semo)

        fire_in(0, ins[0])
        fire_in(1, ins[1])

        def tri_body(j, _):
            for t in range(NIN):
                process(NIN * j + t, t)
            return 0

        lax.fori_loop(0, n_main // NIN, tri_body, 0)

        for t in range(tail):
            process(jnp.int32(n_main + t), t)

        for h in range(2 * n_blocks - NOUT, 2 * n_blocks):
            drain_half(outs[h % NOUT])

    return k


def kernel(x, pos_table):
    B, S, D = x.shape
    k = _sc_add_kernel(B, S, D, R=8, n_workers=32)
    return k(x, pos_table[:S])
